# Initial kernel scaffold; baseline (speedup 1.0000x reference)
import functools

import jax
import jax.numpy as jnp
from jax import lax
from jax.experimental import pallas as pl
from jax.experimental.pallas import tpu as pltpu
from jax.experimental.pallas import tpu_sc as plsc

N_NODES = 10000
N_PAD = 10240
IN_DIM = 128
HID_DIM = 256
OUT_DIM = 128
N_EDGES = 320000

NW = 32
CPT = 4
ECHUNK = 8000
EPW = N_EDGES // NW

_MESH = plsc.VectorSubcoreMesh(core_axis_name="c", subcore_axis_name="s")


def _wid():
    return lax.axis_index("s") * 2 + lax.axis_index("c")


def _pack_body(e_ref, o_ref):
    s = e_ref[0, :]
    d = e_ref[1, :]
    o_ref[...] = s + (d << 16)


def _pack_edges(ei):
    nb = 32000
    return pl.pallas_call(
        _pack_body,
        grid=(N_EDGES // nb,),
        in_specs=[pl.BlockSpec((2, nb), lambda i: (0, i))],
        out_specs=pl.BlockSpec((nb,), lambda i: (i,)),
        out_shape=jax.ShapeDtypeStruct((N_EDGES,), jnp.int32),
    )(ei)


@functools.partial(
    pl.kernel,
    mesh=_MESH,
    out_type=jax.ShapeDtypeStruct((NW, N_PAD), jnp.float32),
    scratch_types=[
        pltpu.VMEM((EPW,), jnp.int32),
        pltpu.VMEM((N_PAD,), jnp.float32),
    ],
)
def _hist_sc(ep_hbm, out_hbm, ebuf, acc):
    wid = _wid()
    zero16 = jnp.zeros((16,), jnp.float32)

    def zbody(i, _):
        acc[pl.ds(i * 16, 16)] = zero16
        return 0

    lax.fori_loop(0, N_PAD // 16, zbody, 0)
    base = pl.multiple_of(wid * EPW, 8)
    pltpu.sync_copy(ep_hbm.at[pl.ds(base, EPW)], ebuf)
    ones = jnp.full((16,), 1.0, jnp.float32)

    def body(k, _):
        epv = ebuf[pl.ds(k * 16, 16)]
        d16 = lax.shift_right_logical(epv, 16)
        plsc.addupdate_scatter(acc, [d16], ones)
        return 0

    lax.fori_loop(0, EPW // 16, body, 0)
    pltpu.sync_copy(acc, out_hbm.at[wid])


def _dis_body(h_ref, o_ref):
    deg = jnp.sum(h_ref[...], axis=0) + 1.0
    o_ref[...] = 1.0 / jnp.sqrt(deg)


def _dis(hist):
    return pl.pallas_call(
        _dis_body,
        out_shape=jax.ShapeDtypeStruct((N_PAD,), jnp.float32),
    )(hist)


def _mm_body(w_ref, t_ref, dis_ref, o_ref):
    prod = lax.dot_general(
        w_ref[...],
        t_ref[...],
        (((1,), (0,)), ((), ())),
        preferred_element_type=jnp.float32,
        precision=lax.Precision.HIGHEST,
    )
    o_ref[...] = prod * dis_ref[...][None, :]


def _mm_scale(wt, t, dis):
    mout, kdim = wt.shape
    nb = 2048
    return pl.pallas_call(
        _mm_body,
        grid=(N_PAD // nb,),
        in_specs=[
            pl.BlockSpec((mout, kdim), lambda i: (0, 0)),
            pl.BlockSpec((kdim, nb), lambda i: (0, i)),
            pl.BlockSpec((nb,), lambda i: (i,)),
        ],
        out_specs=pl.BlockSpec((mout, nb), lambda i: (0, i)),
        out_shape=jax.ShapeDtypeStruct((mout, N_PAD), jnp.float32),
    )(wt, t, dis)


@functools.partial(
    pl.kernel,
    mesh=_MESH,
    out_type=jax.ShapeDtypeStruct((NW * CPT, N_PAD), jnp.float32),
    scratch_types=[
        pltpu.VMEM((CPT, N_PAD), jnp.float32),
        pltpu.VMEM((CPT, N_PAD), jnp.float32),
        pltpu.VMEM((ECHUNK,), jnp.int32),
    ],
)
def _agg_sc(tbl_hbm, ep_hbm, out_hbm, tbl_v, acc_v, ebuf):
    wid = _wid()
    c0 = wid * CPT
    pltpu.sync_copy(tbl_hbm.at[pl.ds(c0, CPT)], tbl_v)
    zero16 = jnp.zeros((16,), jnp.float32)

    def zbody(i, _):
        for r in range(CPT):
            acc_v[r, pl.ds(i * 16, 16)] = zero16
        return 0

    lax.fori_loop(0, N_PAD // 16, zbody, 0)
    cvecs = [jnp.full((16,), c, jnp.int32) for c in range(CPT)]
    m16 = jnp.full((16,), 0xFFFF, jnp.int32)

    def chunk(ci, _):
        off = pl.multiple_of(ci * ECHUNK, 8)
        pltpu.sync_copy(ep_hbm.at[pl.ds(off, ECHUNK)], ebuf)

        def grp(k, _):
            epv = ebuf[pl.ds(k * 16, 16)]
            s16 = lax.bitwise_and(epv, m16)
            d16 = lax.shift_right_logical(epv, 16)
            for c in range(CPT):
                v = plsc.load_gather(tbl_v, [cvecs[c], s16])
                plsc.addupdate_scatter(acc_v, [cvecs[c], d16], v)
            return 0

        lax.fori_loop(0, ECHUNK // 16, grp, 0)
        return 0

    lax.fori_loop(0, N_EDGES // ECHUNK, chunk, 0)
    pltpu.sync_copy(acc_v, out_hbm.at[pl.ds(c0, CPT)])


def _bn_body(a_ref, b_ref, hs_ref, dis_ref, bias_ref, g_ref, be_ref, z_ref):
    acc = jnp.concatenate([a_ref[...], b_ref[...]], axis=0)
    a = dis_ref[...][None, :] * (acc + hs_ref[...]) + bias_ref[...][:, None]
    col = lax.broadcasted_iota(jnp.int32, (HID_DIM, N_PAD), 1)
    m = (col < N_NODES).astype(jnp.float32)
    am = a * m
    s1 = jnp.sum(am, axis=1)
    s2 = jnp.sum(am * am, axis=1)
    mean = s1 / N_NODES
    var = s2 / N_NODES - mean * mean
    rstd = 1.0 / jnp.sqrt(var + 1e-5)
    zn = (a - mean[:, None]) * rstd[:, None] * g_ref[...][:, None] + be_ref[...][:, None]
    z_ref[...] = jnp.maximum(zn, 0.0)


def _bn_relu(acc_a, acc_b, h1s, dis, b1, gamma1, beta1):
    return pl.pallas_call(
        _bn_body,
        out_shape=jax.ShapeDtypeStruct((HID_DIM, N_PAD), jnp.float32),
    )(acc_a, acc_b, h1s, dis, b1, gamma1, beta1)


def _final_body(acc_ref, hs_ref, dis_ref, bias_ref, o_ref):
    v = dis_ref[...][None, :] * (acc_ref[...] + hs_ref[...]) + bias_ref[...][:, None]
    o_ref[...] = v.T


def _final(acc, hs, dis, b2):
    nb = 2048
    return pl.pallas_call(
        _final_body,
        grid=(N_PAD // nb,),
        in_specs=[
            pl.BlockSpec((OUT_DIM, nb), lambda i: (0, i)),
            pl.BlockSpec((OUT_DIM, nb), lambda i: (0, i)),
            pl.BlockSpec((nb,), lambda i: (i,)),
            pl.BlockSpec((OUT_DIM,), lambda i: (0,)),
        ],
        out_specs=pl.BlockSpec((nb, OUT_DIM), lambda i: (i, 0)),
        out_shape=jax.ShapeDtypeStruct((N_PAD, OUT_DIM), jnp.float32),
    )(acc, hs, dis, b2)


def kernel(x, edge_index, W1, b1, gamma1, beta1, W2, b2):
    ei = edge_index.astype(jnp.int32)
    ep = _pack_edges(ei)
    hist = _hist_sc(ep)
    dis = _dis(hist)

    xt = jnp.zeros((IN_DIM, N_PAD), jnp.float32).at[:, :N_NODES].set(x.T)
    h1s = _mm_scale(W1.T, xt, dis)

    acc1a = _agg_sc(h1s[:128], ep)
    acc1b = _agg_sc(h1s[128:], ep)
    z = _bn_relu(acc1a, acc1b, h1s, dis, b1, gamma1, beta1)

    h2s = _mm_scale(W2.T, z, dis)
    acc2 = _agg_sc(h2s, ep)
    out = _final(acc2, h2s, dis, b2)
    return out[:N_NODES]


# speedup vs baseline: 6.1329x; 6.1329x over previous
import functools

import jax
import jax.numpy as jnp
from jax import lax
from jax.experimental import pallas as pl
from jax.experimental.pallas import tpu as pltpu
from jax.experimental.pallas import tpu_sc as plsc

N_NODES = 10000
N_PAD = 10240
IN_DIM = 128
HID_DIM = 256
OUT_DIM = 128
N_EDGES = 320000

NW = 32
CPT = 4
ECHUNK = 8000
EPW = N_EDGES // NW

_MESH = plsc.VectorSubcoreMesh(
    core_axis_name="c", subcore_axis_name="s", num_cores=2, num_subcores=16
)
_SC_PARAMS = pltpu.CompilerParams(
    needs_layout_passes=False, use_tc_tiling_on_sc=False
)


def _wid():
    return lax.axis_index("s") * 2 + lax.axis_index("c")


def _pack_body(e_ref, o_ref):
    s = e_ref[0, :]
    d = e_ref[1, :]
    o_ref[...] = s + (d << 16)


def _pack_edges(ei):
    return pl.pallas_call(
        _pack_body,
        out_shape=jax.ShapeDtypeStruct((N_EDGES,), jnp.int32),
    )(ei)


@functools.partial(
    pl.kernel,
    mesh=_MESH,
    compiler_params=_SC_PARAMS,
    out_type=jax.ShapeDtypeStruct((NW, N_PAD), jnp.float32),
    scratch_types=[
        pltpu.VMEM((EPW,), jnp.int32),
        pltpu.VMEM((N_PAD,), jnp.float32),
    ],
)
def _hist_sc(ep_hbm, out_hbm, ebuf, acc):
    wid = _wid()
    zero16 = jnp.zeros((16,), jnp.float32)

    def zbody(i, _):
        acc[pl.ds(i * 16, 16)] = zero16
        return 0

    lax.fori_loop(0, N_PAD // 16, zbody, 0)
    base = pl.multiple_of(wid * EPW, 8)
    pltpu.sync_copy(ep_hbm.at[pl.ds(base, EPW)], ebuf)
    ones = jnp.full((16,), 1.0, jnp.float32)

    def body(k, _):
        epv = ebuf[pl.ds(k * 16, 16)]
        d16 = lax.shift_right_logical(epv, 16)
        plsc.addupdate_scatter(acc, [d16], ones)
        return 0

    lax.fori_loop(0, EPW // 16, body, 0)
    pltpu.sync_copy(acc, out_hbm.at[wid])


def _dis_body(h_ref, o_ref):
    deg = jnp.sum(h_ref[...], axis=0) + 1.0
    o_ref[...] = 1.0 / jnp.sqrt(deg)


def _dis(hist):
    return pl.pallas_call(
        _dis_body,
        out_shape=jax.ShapeDtypeStruct((N_PAD,), jnp.float32),
    )(hist)


def _mm_body(w_ref, t_ref, dis_ref, o_ref):
    prod = lax.dot_general(
        w_ref[...],
        t_ref[...],
        (((1,), (0,)), ((), ())),
        preferred_element_type=jnp.float32,
        precision=lax.Precision.HIGHEST,
    )
    o_ref[...] = prod * dis_ref[...][None, :]


def _mm_scale(wt, t, dis):
    mout, kdim = wt.shape
    nb = 2048
    return pl.pallas_call(
        _mm_body,
        grid=(N_PAD // nb,),
        in_specs=[
            pl.BlockSpec((mout, kdim), lambda i: (0, 0)),
            pl.BlockSpec((kdim, nb), lambda i: (0, i)),
            pl.BlockSpec((nb,), lambda i: (i,)),
        ],
        out_specs=pl.BlockSpec((mout, nb), lambda i: (0, i)),
        out_shape=jax.ShapeDtypeStruct((mout, N_PAD), jnp.float32),
    )(wt, t, dis)


@functools.partial(
    pl.kernel,
    mesh=_MESH,
    compiler_params=_SC_PARAMS,
    out_type=jax.ShapeDtypeStruct((NW * CPT, N_PAD), jnp.float32),
    scratch_types=[
        pltpu.VMEM((CPT, N_PAD), jnp.float32),
        pltpu.VMEM((CPT, N_PAD), jnp.float32),
        pltpu.VMEM((ECHUNK,), jnp.int32),
    ],
)
def _agg_sc(tbl_hbm, ep_hbm, out_hbm, tbl_v, acc_v, ebuf):
    wid = _wid()
    c0 = wid * CPT
    pltpu.sync_copy(tbl_hbm.at[pl.ds(c0, CPT)], tbl_v)
    zero16 = jnp.zeros((16,), jnp.float32)

    def zbody(i, _):
        for r in range(CPT):
            acc_v[r, pl.ds(i * 16, 16)] = zero16
        return 0

    lax.fori_loop(0, N_PAD // 16, zbody, 0)
    cvecs = [jnp.full((16,), c, jnp.int32) for c in range(CPT)]
    m16 = jnp.full((16,), 0xFFFF, jnp.int32)

    def chunk(ci, _):
        off = pl.multiple_of(ci * ECHUNK, 8)
        pltpu.sync_copy(ep_hbm.at[pl.ds(off, ECHUNK)], ebuf)

        def grp(k, _):
            epv = ebuf[pl.ds(k * 16, 16)]
            s16 = lax.bitwise_and(epv, m16)
            d16 = lax.shift_right_logical(epv, 16)
            for c in range(CPT):
                v = plsc.load_gather(tbl_v, [cvecs[c], s16])
                plsc.addupdate_scatter(acc_v, [cvecs[c], d16], v)
            return 0

        lax.fori_loop(0, ECHUNK // 16, grp, 0)
        return 0

    lax.fori_loop(0, N_EDGES // ECHUNK, chunk, 0)
    pltpu.sync_copy(acc_v, out_hbm.at[pl.ds(c0, CPT)])


def _bn_body(a_ref, b_ref, hs_ref, dis_ref, bias_ref, g_ref, be_ref, z_ref):
    acc = jnp.concatenate([a_ref[...], b_ref[...]], axis=0)
    a = dis_ref[...][None, :] * (acc + hs_ref[...]) + bias_ref[...][:, None]
    col = lax.broadcasted_iota(jnp.int32, (HID_DIM, N_PAD), 1)
    m = (col < N_NODES).astype(jnp.float32)
    am = a * m
    s1 = jnp.sum(am, axis=1)
    s2 = jnp.sum(am * am, axis=1)
    mean = s1 / N_NODES
    var = s2 / N_NODES - mean * mean
    rstd = 1.0 / jnp.sqrt(var + 1e-5)
    zn = (a - mean[:, None]) * rstd[:, None] * g_ref[...][:, None] + be_ref[...][:, None]
    z_ref[...] = jnp.maximum(zn, 0.0)


def _bn_relu(acc_a, acc_b, h1s, dis, b1, gamma1, beta1):
    return pl.pallas_call(
        _bn_body,
        out_shape=jax.ShapeDtypeStruct((HID_DIM, N_PAD), jnp.float32),
    )(acc_a, acc_b, h1s, dis, b1, gamma1, beta1)


def _final_body(acc_ref, hs_ref, dis_ref, bias_ref, o_ref):
    v = dis_ref[...][None, :] * (acc_ref[...] + hs_ref[...]) + bias_ref[...][:, None]
    o_ref[...] = v.T


def _final(acc, hs, dis, b2):
    nb = 2048
    return pl.pallas_call(
        _final_body,
        grid=(N_PAD // nb,),
        in_specs=[
            pl.BlockSpec((OUT_DIM, nb), lambda i: (0, i)),
            pl.BlockSpec((OUT_DIM, nb), lambda i: (0, i)),
            pl.BlockSpec((nb,), lambda i: (i,)),
            pl.BlockSpec((OUT_DIM,), lambda i: (0,)),
        ],
        out_specs=pl.BlockSpec((nb, OUT_DIM), lambda i: (i, 0)),
        out_shape=jax.ShapeDtypeStruct((N_PAD, OUT_DIM), jnp.float32),
    )(acc, hs, dis, b2)


def kernel(x, edge_index, W1, b1, gamma1, beta1, W2, b2):
    ei = edge_index.astype(jnp.int32)
    ep = _pack_edges(ei)
    hist = _hist_sc(ep)
    dis = _dis(hist)

    xt = jnp.zeros((IN_DIM, N_PAD), jnp.float32).at[:, :N_NODES].set(x.T)
    h1s = _mm_scale(W1.T, xt, dis)

    acc1a = _agg_sc(h1s[:128], ep)
    acc1b = _agg_sc(h1s[128:], ep)
    z = _bn_relu(acc1a, acc1b, h1s, dis, b1, gamma1, beta1)

    h2s = _mm_scale(W2.T, z, dis)
    acc2 = _agg_sc(h2s, ep)
    out = _final(acc2, h2s, dis, b2)
    return out[:N_NODES]


# speedup vs baseline: 6.1882x; 1.0090x over previous
import functools

import jax
import jax.numpy as jnp
from jax import lax
from jax.experimental import pallas as pl
from jax.experimental.pallas import tpu as pltpu
from jax.experimental.pallas import tpu_sc as plsc

N_NODES = 10000
N_PAD = 10240
IN_DIM = 128
HID_DIM = 256
OUT_DIM = 128
N_EDGES = 320000

NW = 32
CPT = 4
ECHUNK = 8000
UNROLL = 10
EPW = N_EDGES // NW

_MESH = plsc.VectorSubcoreMesh(
    core_axis_name="c", subcore_axis_name="s", num_cores=2, num_subcores=16
)
_SC_PARAMS = pltpu.CompilerParams(
    needs_layout_passes=False, use_tc_tiling_on_sc=False
)


def _wid():
    return lax.axis_index("s") * 2 + lax.axis_index("c")


def _pack_body(e_ref, o_ref):
    s = e_ref[0, :]
    d = e_ref[1, :]
    o_ref[...] = s + (d << 16)


def _pack_edges(ei):
    return pl.pallas_call(
        _pack_body,
        out_shape=jax.ShapeDtypeStruct((N_EDGES,), jnp.int32),
    )(ei)


@functools.partial(
    pl.kernel,
    mesh=_MESH,
    compiler_params=_SC_PARAMS,
    out_type=jax.ShapeDtypeStruct((NW, N_PAD), jnp.float32),
    scratch_types=[
        pltpu.VMEM((EPW,), jnp.int32),
        pltpu.VMEM((N_PAD,), jnp.float32),
    ],
)
def _hist_sc(ep_hbm, out_hbm, ebuf, acc):
    wid = _wid()
    zero16 = jnp.zeros((16,), jnp.float32)

    def zbody(i, _):
        acc[pl.ds(i * 16, 16)] = zero16
        return 0

    lax.fori_loop(0, N_PAD // 16, zbody, 0)
    base = pl.multiple_of(wid * EPW, 8)
    pltpu.sync_copy(ep_hbm.at[pl.ds(base, EPW)], ebuf)
    ones = jnp.full((16,), 1.0, jnp.float32)

    def body(k, _):
        epv = ebuf[pl.ds(k * 16, 16)]
        d16 = lax.shift_right_logical(epv, 16)
        plsc.addupdate_scatter(acc, [d16], ones)
        return 0

    lax.fori_loop(0, EPW // 16, body, 0)
    pltpu.sync_copy(acc, out_hbm.at[wid])


def _dis_body(h_ref, o_ref):
    deg = jnp.sum(h_ref[...], axis=0) + 1.0
    o_ref[...] = 1.0 / jnp.sqrt(deg)


def _dis(hist):
    return pl.pallas_call(
        _dis_body,
        out_shape=jax.ShapeDtypeStruct((N_PAD,), jnp.float32),
    )(hist)


def _mm_body(w_ref, t_ref, dis_ref, o_ref):
    prod = lax.dot_general(
        w_ref[...],
        t_ref[...],
        (((1,), (0,)), ((), ())),
        preferred_element_type=jnp.float32,
        precision=lax.Precision.HIGHEST,
    )
    o_ref[...] = prod * dis_ref[...][None, :]


def _mm_scale(wt, t, dis):
    mout, kdim = wt.shape
    nb = 2048
    return pl.pallas_call(
        _mm_body,
        grid=(N_PAD // nb,),
        in_specs=[
            pl.BlockSpec((mout, kdim), lambda i: (0, 0)),
            pl.BlockSpec((kdim, nb), lambda i: (0, i)),
            pl.BlockSpec((nb,), lambda i: (i,)),
        ],
        out_specs=pl.BlockSpec((mout, nb), lambda i: (0, i)),
        out_shape=jax.ShapeDtypeStruct((mout, N_PAD), jnp.float32),
    )(wt, t, dis)


@functools.partial(
    pl.kernel,
    mesh=_MESH,
    compiler_params=_SC_PARAMS,
    out_type=jax.ShapeDtypeStruct((NW * CPT, N_PAD), jnp.float32),
    scratch_types=[
        pltpu.VMEM((CPT, N_PAD), jnp.float32),
        pltpu.VMEM((CPT, N_PAD), jnp.float32),
        pltpu.VMEM((ECHUNK,), jnp.int32),
    ],
)
def _agg_sc(tbl_hbm, ep_hbm, out_hbm, tbl_v, acc_v, ebuf):
    wid = _wid()
    c0 = wid * CPT
    pltpu.sync_copy(tbl_hbm.at[pl.ds(c0, CPT)], tbl_v)
    zero16 = jnp.zeros((16,), jnp.float32)

    def zbody(i, _):
        for r in range(CPT):
            acc_v[r, pl.ds(i * 16, 16)] = zero16
        return 0

    lax.fori_loop(0, N_PAD // 16, zbody, 0)
    cvecs = [jnp.full((16,), c, jnp.int32) for c in range(CPT)]
    m16 = jnp.full((16,), 0xFFFF, jnp.int32)

    def chunk(ci, _):
        off = pl.multiple_of(ci * ECHUNK, 8)
        pltpu.sync_copy(ep_hbm.at[pl.ds(off, ECHUNK)], ebuf)

        def grp(k, _):
            for u in range(UNROLL):
                epv = ebuf[pl.ds((k * UNROLL + u) * 16, 16)]
                s16 = lax.bitwise_and(epv, m16)
                d16 = lax.shift_right_logical(epv, 16)
                for c in range(CPT):
                    v = plsc.load_gather(tbl_v, [cvecs[c], s16])
                    plsc.addupdate_scatter(acc_v, [cvecs[c], d16], v)
            return 0

        lax.fori_loop(0, ECHUNK // 16 // UNROLL, grp, 0)
        return 0

    lax.fori_loop(0, N_EDGES // ECHUNK, chunk, 0)
    pltpu.sync_copy(acc_v, out_hbm.at[pl.ds(c0, CPT)])


def _bn_body(a_ref, b_ref, hs_ref, dis_ref, bias_ref, g_ref, be_ref, z_ref):
    acc = jnp.concatenate([a_ref[...], b_ref[...]], axis=0)
    a = dis_ref[...][None, :] * (acc + hs_ref[...]) + bias_ref[...][:, None]
    col = lax.broadcasted_iota(jnp.int32, (HID_DIM, N_PAD), 1)
    m = (col < N_NODES).astype(jnp.float32)
    am = a * m
    s1 = jnp.sum(am, axis=1)
    s2 = jnp.sum(am * am, axis=1)
    mean = s1 / N_NODES
    var = s2 / N_NODES - mean * mean
    rstd = 1.0 / jnp.sqrt(var + 1e-5)
    zn = (a - mean[:, None]) * rstd[:, None] * g_ref[...][:, None] + be_ref[...][:, None]
    z_ref[...] = jnp.maximum(zn, 0.0)


def _bn_relu(acc_a, acc_b, h1s, dis, b1, gamma1, beta1):
    return pl.pallas_call(
        _bn_body,
        out_shape=jax.ShapeDtypeStruct((HID_DIM, N_PAD), jnp.float32),
    )(acc_a, acc_b, h1s, dis, b1, gamma1, beta1)


def _final_body(acc_ref, hs_ref, dis_ref, bias_ref, o_ref):
    v = dis_ref[...][None, :] * (acc_ref[...] + hs_ref[...]) + bias_ref[...][:, None]
    o_ref[...] = v.T


def _final(acc, hs, dis, b2):
    nb = 2048
    return pl.pallas_call(
        _final_body,
        grid=(N_PAD // nb,),
        in_specs=[
            pl.BlockSpec((OUT_DIM, nb), lambda i: (0, i)),
            pl.BlockSpec((OUT_DIM, nb), lambda i: (0, i)),
            pl.BlockSpec((nb,), lambda i: (i,)),
            pl.BlockSpec((OUT_DIM,), lambda i: (0,)),
        ],
        out_specs=pl.BlockSpec((nb, OUT_DIM), lambda i: (i, 0)),
        out_shape=jax.ShapeDtypeStruct((N_PAD, OUT_DIM), jnp.float32),
    )(acc, hs, dis, b2)


def kernel(x, edge_index, W1, b1, gamma1, beta1, W2, b2):
    ei = edge_index.astype(jnp.int32)
    ep = _pack_edges(ei)
    hist = _hist_sc(ep)
    dis = _dis(hist)

    xt = jnp.zeros((IN_DIM, N_PAD), jnp.float32).at[:, :N_NODES].set(x.T)
    h1s = _mm_scale(W1.T, xt, dis)

    acc1a = _agg_sc(h1s[:128], ep)
    acc1b = _agg_sc(h1s[128:], ep)
    z = _bn_relu(acc1a, acc1b, h1s, dis, b1, gamma1, beta1)

    h2s = _mm_scale(W2.T, z, dis)
    acc2 = _agg_sc(h2s, ep)
    out = _final(acc2, h2s, dis, b2)
    return out[:N_NODES]


# speedup vs baseline: 12.8918x; 2.0833x over previous
import functools

import jax
import jax.numpy as jnp
from jax import lax
from jax.experimental import pallas as pl
from jax.experimental.pallas import tpu as pltpu
from jax.experimental.pallas import tpu_sc as plsc

N_NODES = 10000
N_PAD = 10240
IN_DIM = 128
HID_DIM = 256
OUT_DIM = 128
N_EDGES = 320000

NW = 32
CPT = 4
ECHUNK = 8000
UNROLL = 4
EPW = N_EDGES // NW

_MESH = plsc.VectorSubcoreMesh(
    core_axis_name="c", subcore_axis_name="s", num_cores=2, num_subcores=16
)
_SC_PARAMS = pltpu.CompilerParams(
    needs_layout_passes=False, use_tc_tiling_on_sc=False
)


def _wid():
    return lax.axis_index("s") * 2 + lax.axis_index("c")


def _pack_body(e_ref, o_ref):
    s = e_ref[0, :]
    d = e_ref[1, :]
    o_ref[...] = s + (d << 16)


def _pack_edges(ei):
    return pl.pallas_call(
        _pack_body,
        out_shape=jax.ShapeDtypeStruct((N_EDGES,), jnp.int32),
    )(ei)


@functools.partial(
    pl.kernel,
    mesh=_MESH,
    compiler_params=_SC_PARAMS,
    out_type=jax.ShapeDtypeStruct((NW, N_PAD), jnp.float32),
    scratch_types=[
        pltpu.VMEM((EPW,), jnp.int32),
        pltpu.VMEM((N_PAD,), jnp.float32),
    ],
)
def _hist_sc(ep_hbm, out_hbm, ebuf, acc):
    wid = _wid()
    zero16 = jnp.zeros((16,), jnp.float32)

    def zbody(i, _):
        acc[pl.ds(i * 16, 16)] = zero16
        return 0

    lax.fori_loop(0, N_PAD // 16, zbody, 0)
    base = pl.multiple_of(wid * EPW, 8)
    pltpu.sync_copy(ep_hbm.at[pl.ds(base, EPW)], ebuf)
    ones = jnp.full((16,), 1.0, jnp.float32)

    def body(k, _):
        epv = ebuf[pl.ds(k * 16, 16)]
        d16 = lax.shift_right_logical(epv, 16)
        plsc.addupdate_scatter(acc, [d16], ones)
        return 0

    lax.fori_loop(0, EPW // 16, body, 0)
    pltpu.sync_copy(acc, out_hbm.at[wid])


def _dis_body(h_ref, o_ref):
    deg = jnp.sum(h_ref[...], axis=0) + 1.0
    o_ref[...] = 1.0 / jnp.sqrt(deg)


def _dis(hist):
    return pl.pallas_call(
        _dis_body,
        out_shape=jax.ShapeDtypeStruct((N_PAD,), jnp.float32),
    )(hist)


def _mm_body(w_ref, t_ref, dis_ref, o_ref):
    prod = lax.dot_general(
        w_ref[...],
        t_ref[...],
        (((1,), (0,)), ((), ())),
        preferred_element_type=jnp.float32,
        precision=lax.Precision.HIGHEST,
    )
    o_ref[...] = prod * dis_ref[...][None, :]


def _mm_scale(wt, t, dis):
    mout, kdim = wt.shape
    nb = 2048
    return pl.pallas_call(
        _mm_body,
        grid=(N_PAD // nb,),
        in_specs=[
            pl.BlockSpec((mout, kdim), lambda i: (0, 0)),
            pl.BlockSpec((kdim, nb), lambda i: (0, i)),
            pl.BlockSpec((nb,), lambda i: (i,)),
        ],
        out_specs=pl.BlockSpec((mout, nb), lambda i: (0, i)),
        out_shape=jax.ShapeDtypeStruct((mout, N_PAD), jnp.float32),
    )(wt, t, dis)


@functools.partial(
    pl.kernel,
    mesh=_MESH,
    compiler_params=_SC_PARAMS,
    out_type=jax.ShapeDtypeStruct((NW * CPT, N_PAD), jnp.float32),
    scratch_types=[
        pltpu.VMEM((CPT, N_PAD), jnp.float32),
        pltpu.VMEM((CPT, N_PAD), jnp.float32),
        pltpu.VMEM((ECHUNK,), jnp.int32),
    ],
)
def _agg_sc(tbl_hbm, ep_hbm, out_hbm, tbl_v, acc_v, ebuf):
    wid = _wid()
    c0 = wid * CPT
    pltpu.sync_copy(tbl_hbm.at[pl.ds(c0, CPT)], tbl_v)
    zero16 = jnp.zeros((16,), jnp.float32)

    def zbody(i, _):
        for r in range(CPT):
            acc_v[r, pl.ds(i * 16, 16)] = zero16
        return 0

    lax.fori_loop(0, N_PAD // 16, zbody, 0)
    cvecs = [jnp.full((16,), c, jnp.int32) for c in range(CPT)]
    m16 = jnp.full((16,), 0xFFFF, jnp.int32)

    def chunk(ci, _):
        off = pl.multiple_of(ci * ECHUNK, 8)
        pltpu.sync_copy(ep_hbm.at[pl.ds(off, ECHUNK)], ebuf)

        def grp(k, _):
            svecs = []
            dvecs = []
            for u in range(UNROLL):
                epv = ebuf[pl.ds((k * UNROLL + u) * 16, 16)]
                svecs.append(lax.bitwise_and(epv, m16))
                dvecs.append(lax.shift_right_logical(epv, 16))
            vals = [
                plsc.load_gather(tbl_v, [cvecs[c], svecs[u]])
                for u in range(UNROLL)
                for c in range(CPT)
            ]
            for u in range(UNROLL):
                for c in range(CPT):
                    plsc.addupdate_scatter(
                        acc_v, [cvecs[c], dvecs[u]], vals[u * CPT + c]
                    )
            return 0

        lax.fori_loop(0, ECHUNK // 16 // UNROLL, grp, 0)
        return 0

    lax.fori_loop(0, N_EDGES // ECHUNK, chunk, 0)
    pltpu.sync_copy(acc_v, out_hbm.at[pl.ds(c0, CPT)])


def _bn_body(a_ref, b_ref, hs_ref, dis_ref, bias_ref, g_ref, be_ref, z_ref):
    acc = jnp.concatenate([a_ref[...], b_ref[...]], axis=0)
    a = dis_ref[...][None, :] * (acc + hs_ref[...]) + bias_ref[...][:, None]
    col = lax.broadcasted_iota(jnp.int32, (HID_DIM, N_PAD), 1)
    m = (col < N_NODES).astype(jnp.float32)
    am = a * m
    s1 = jnp.sum(am, axis=1)
    s2 = jnp.sum(am * am, axis=1)
    mean = s1 / N_NODES
    var = s2 / N_NODES - mean * mean
    rstd = 1.0 / jnp.sqrt(var + 1e-5)
    zn = (a - mean[:, None]) * rstd[:, None] * g_ref[...][:, None] + be_ref[...][:, None]
    z_ref[...] = jnp.maximum(zn, 0.0)


def _bn_relu(acc_a, acc_b, h1s, dis, b1, gamma1, beta1):
    return pl.pallas_call(
        _bn_body,
        out_shape=jax.ShapeDtypeStruct((HID_DIM, N_PAD), jnp.float32),
    )(acc_a, acc_b, h1s, dis, b1, gamma1, beta1)


def _final_body(acc_ref, hs_ref, dis_ref, bias_ref, o_ref):
    v = dis_ref[...][None, :] * (acc_ref[...] + hs_ref[...]) + bias_ref[...][:, None]
    o_ref[...] = v.T


def _final(acc, hs, dis, b2):
    nb = 2048
    return pl.pallas_call(
        _final_body,
        grid=(N_PAD // nb,),
        in_specs=[
            pl.BlockSpec((OUT_DIM, nb), lambda i: (0, i)),
            pl.BlockSpec((OUT_DIM, nb), lambda i: (0, i)),
            pl.BlockSpec((nb,), lambda i: (i,)),
            pl.BlockSpec((OUT_DIM,), lambda i: (0,)),
        ],
        out_specs=pl.BlockSpec((nb, OUT_DIM), lambda i: (i, 0)),
        out_shape=jax.ShapeDtypeStruct((N_PAD, OUT_DIM), jnp.float32),
    )(acc, hs, dis, b2)


def kernel(x, edge_index, W1, b1, gamma1, beta1, W2, b2):
    ei = edge_index.astype(jnp.int32)
    ep = _pack_edges(ei)
    hist = _hist_sc(ep)
    dis = _dis(hist)

    xt = jnp.zeros((IN_DIM, N_PAD), jnp.float32).at[:, :N_NODES].set(x.T)
    h1s = _mm_scale(W1.T, xt, dis)

    acc1a = _agg_sc(h1s[:128], ep)
    acc1b = _agg_sc(h1s[128:], ep)
    z = _bn_relu(acc1a, acc1b, h1s, dis, b1, gamma1, beta1)

    h2s = _mm_scale(W2.T, z, dis)
    acc2 = _agg_sc(h2s, ep)
    out = _final(acc2, h2s, dis, b2)
    return out[:N_NODES]
